# baseline (speedup 1.0000x reference)
import functools

import jax
import jax.numpy as jnp
from jax import lax
from jax.experimental import pallas as pl
from jax.experimental.pallas import tpu as pltpu
from jax.experimental.pallas import tpu_sc as plsc

_ROWS = 1_000_000
_DIM = 64
_NC = 2
_NS = 16
_NW = _NC * _NS
_CHUNK = 400
_NCHUNKS = _ROWS // _CHUNK
_MAX_PER_W = -(-_NCHUNKS // _NW)

_mesh = plsc.VectorSubcoreMesh(core_axis_name="c", subcore_axis_name="s")


@functools.partial(
    pl.kernel,
    out_type=jax.ShapeDtypeStruct((_ROWS, _DIM), jnp.float32),
    mesh=_mesh,
    scratch_types=[
        pltpu.VMEM((2, _CHUNK, _DIM), jnp.float32),
        pltpu.SemaphoreType.DMA,
        pltpu.SemaphoreType.DMA,
        pltpu.SemaphoreType.DMA,
        pltpu.SemaphoreType.DMA,
    ],
)
def _sc_copy(in_hbm, out_hbm, buf, in_sem0, in_sem1, out_sem0, out_sem1):
    wid = lax.axis_index("s") * _NC + lax.axis_index("c")
    in_sems = (in_sem0, in_sem1)
    out_sems = (out_sem0, out_sem1)

    def in_copy(g, b):
        base = pl.multiple_of((wid + g * _NW) * _CHUNK, 8)
        return pltpu.make_async_copy(
            in_hbm.at[pl.ds(base, _CHUNK), :], buf.at[b], in_sems[b]
        )

    def out_copy(g, b):
        base = pl.multiple_of((wid + g * _NW) * _CHUNK, 8)
        return pltpu.make_async_copy(
            buf.at[b], out_hbm.at[pl.ds(base, _CHUNK), :], out_sems[b]
        )

    def exists(g):
        return wid + g * _NW < _NCHUNKS

    pl.when(exists(0))(lambda: in_copy(0, 0).start())
    for g in range(_MAX_PER_W):
        b = g % 2
        if g >= 1:
            pl.when(exists(g - 1))(lambda g=g, b=b: out_copy(g - 1, 1 - b).wait())
        if g + 1 < _MAX_PER_W:
            pl.when(exists(g + 1))(lambda g=g, b=b: in_copy(g + 1, 1 - b).start())

        @pl.when(exists(g))
        def _(g=g, b=b):
            in_copy(g, b).wait()
            out_copy(g, b).start()

    g_last = _MAX_PER_W - 1
    pl.when(exists(g_last))(lambda: out_copy(g_last, g_last % 2).wait())


def kernel(path_emb):
    return _sc_copy(path_emb)
